# Initial kernel scaffold; baseline (speedup 1.0000x reference)
import jax
import jax.numpy as jnp
from jax.experimental import pallas as pl


def kernel(keypoints, boxes, class_ids, anchor_sizes, anchor_radii):
    raise NotImplementedError("write your pallas kernel here")


# speedup vs baseline: 1.5700x; 1.5700x over previous
import jax
import jax.numpy as jnp
from jax.experimental import pallas as pl
from jax.experimental.pallas import tpu as pltpu

_C = 3
_NCLS = 5
_NEG = 512
_T = 512
_MPAD = 256


def _body(aux_ref, kp_ref, bx_ref, cls_ref, reg_ref):
    kx = kp_ref[0, :, 0:1]
    ky = kp_ref[0, :, 1:2]
    kz = kp_ref[0, :, 2:3]
    neg = kp_ref[0, :, 3:4] > 0.0

    cx = bx_ref[0, 0:1, :]
    cy = bx_ref[0, 1:2, :]
    cz = bx_ref[0, 2:3, :]
    clsf = bx_ref[0, 3:4, :]

    r0 = aux_ref[6, 0]
    r1 = aux_ref[6, 1]
    r2 = aux_ref[6, 2]
    rad = jnp.where(clsf == 0.0, r0, jnp.where(clsf == 1.0, r1, r2))

    dx = kx - cx
    dy = ky - cy
    dz = kz - cz
    dist = jnp.sqrt(dx * dx + dy * dy + dz * dz)
    inr = (dist < rad) & (clsf >= 0.0)

    pos = [jnp.any(inr & (clsf == float(c)), axis=1, keepdims=True)
           for c in range(_C)]
    anyp = pos[0] | pos[1] | pos[2]
    colbg = neg & ~anyp
    colig = ~neg & ~anyp
    cls_cols = [p.astype(jnp.float32) for p in pos + [colbg, colig]]
    cls_ref[0] = jnp.concatenate(cls_cols, axis=1)

    cols = []
    for c in range(_C):
        pf = pos[c].astype(jnp.float32)
        asx = aux_ref[3 + c, 0]
        asy = aux_ref[3 + c, 1]
        asz = aux_ref[3 + c, 2]
        cols.append((aux_ref[c, 0] - kx) * pf)
        cols.append((aux_ref[c, 1] - ky) * pf)
        cols.append((aux_ref[c, 2] - kz) * pf)
        cols.append(((aux_ref[c, 3] - asx) / asx) * pf)
        cols.append(((aux_ref[c, 4] - asy) / asy) * pf)
        cols.append(((aux_ref[c, 5] - asz) / asz) * pf)
        cols.append(aux_ref[c, 6] * pf)
    cols.append(jnp.zeros((kx.shape[0], 7), jnp.float32))
    reg_ref[0] = jnp.concatenate(cols, axis=1)


def kernel(keypoints, boxes, class_ids, anchor_sizes, anchor_radii):
    B, N, _ = keypoints.shape
    nb = boxes.shape[1]
    npad = ((N + _T - 1) // _T) * _T

    neg_inds = jax.random.randint(jax.random.key(1), (B, _NEG), 0, N)
    negmask = jnp.zeros((N,), jnp.float32).at[neg_inds.reshape(-1)].set(1.0)
    negmask = jnp.broadcast_to(negmask[None, :, None], (B, N, 1))

    kp = jnp.concatenate([keypoints, negmask], axis=2)
    kp = jnp.pad(kp, ((0, 0), (0, npad - N), (0, 0)))

    centers = boxes[..., 0:3].transpose(0, 2, 1)
    clsrow = class_ids.astype(jnp.float32)[:, None, :]
    bx = jnp.concatenate([centers, clsrow], axis=1)
    bx = jnp.pad(bx, ((0, 0), (0, 4), (0, _MPAD - nb)), constant_values=-1.0)

    aux = jnp.zeros((8, 8), jnp.float32)
    aux = aux.at[0:3, 0:7].set(boxes.reshape(-1, 7)[0:_C])
    aux = aux.at[3:6, 0:3].set(anchor_sizes)
    aux = aux.at[6, 0:3].set(anchor_radii)

    clsf32, regf32 = pl.pallas_call(
        _body,
        grid=(B, npad // _T),
        in_specs=[
            pl.BlockSpec((8, 8), lambda b, n: (0, 0), memory_space=pltpu.SMEM),
            pl.BlockSpec((1, _T, 4), lambda b, n: (b, n, 0)),
            pl.BlockSpec((1, 8, _MPAD), lambda b, n: (b, 0, 0)),
        ],
        out_specs=[
            pl.BlockSpec((1, _T, _NCLS), lambda b, n: (b, n, 0)),
            pl.BlockSpec((1, _T, 28), lambda b, n: (b, n, 0)),
        ],
        out_shape=[
            jax.ShapeDtypeStruct((B, npad, _NCLS), jnp.float32),
            jax.ShapeDtypeStruct((B, npad, 28), jnp.float32),
        ],
    )(aux, kp, bx)

    targets_cls = clsf32[:, :N, :].astype(bool)
    targets_reg = regf32[:, :N, :].reshape(B, N, 4, 7)
    return targets_cls, targets_reg


# speedup vs baseline: 3.4530x; 2.1994x over previous
import jax
import jax.numpy as jnp
from jax import lax
from jax.experimental import pallas as pl
from jax.experimental.pallas import tpu as pltpu

_C = 3
_NEG = 512
_T = 512
_MPAD = 256


def _body(aux_ref, kp_ref, bxt_ref, clsrow_ref, cls_ref, reg_ref):
    kx = kp_ref[0, 0:1, :]
    ky = kp_ref[0, 1:2, :]
    kz = kp_ref[0, 2:3, :]
    neg = kp_ref[0, 3:4, :]

    cx = bxt_ref[0, :, 0:1]
    cy = bxt_ref[0, :, 1:2]
    cz = bxt_ref[0, :, 2:3]
    clsc = bxt_ref[0, :, 3:4]

    r0 = aux_ref[6, 0]
    r1 = aux_ref[6, 1]
    r2 = aux_ref[6, 2]
    rad = jnp.where(clsc == 0.0, r0, jnp.where(clsc == 1.0, r1, r2))

    dx = cx - kx
    dy = cy - ky
    dz = cz - kz
    dist = jnp.sqrt(dx * dx + dy * dy + dz * dz)
    ind = ((dist < rad) & (clsc >= 0.0)).astype(jnp.float32)

    clsrow = clsrow_ref[0, 0:1, :]
    riota = lax.broadcasted_iota(jnp.int32, (8, _MPAD), 0)
    w = ((riota == clsrow.astype(jnp.int32)) | (riota == 3)).astype(jnp.float32)
    cnt = lax.dot_general(w, ind, (((1,), (0,)), ((), ())),
                          preferred_element_type=jnp.float32)

    pf = [jnp.minimum(cnt[c:c + 1, :], 1.0) for c in range(_C)]
    nanyf = 1.0 - jnp.minimum(cnt[3:4, :], 1.0)
    colbg = neg * nanyf
    colig = (1.0 - neg) * nanyf
    zrow = jnp.zeros_like(kx)
    cls_ref[0] = jnp.concatenate(pf + [colbg, colig, zrow, zrow, zrow], axis=0)

    rows = []
    for c in range(_C):
        asx = aux_ref[3 + c, 0]
        asy = aux_ref[3 + c, 1]
        asz = aux_ref[3 + c, 2]
        rows.append((aux_ref[c, 0] - kx) * pf[c])
        rows.append((aux_ref[c, 1] - ky) * pf[c])
        rows.append((aux_ref[c, 2] - kz) * pf[c])
        rows.append(((aux_ref[c, 3] - asx) / asx) * pf[c])
        rows.append(((aux_ref[c, 4] - asy) / asy) * pf[c])
        rows.append(((aux_ref[c, 5] - asz) / asz) * pf[c])
        rows.append(aux_ref[c, 6] * pf[c])
    rows.append(jnp.zeros((11, kx.shape[1]), jnp.float32))
    reg_ref[0] = jnp.concatenate(rows, axis=0)


def kernel(keypoints, boxes, class_ids, anchor_sizes, anchor_radii):
    B, N, _ = keypoints.shape
    nb = boxes.shape[1]
    npad = ((N + _T - 1) // _T) * _T

    neg_inds = jax.random.randint(jax.random.key(1), (B, _NEG), 0, N)
    negmask = jnp.zeros((N,), jnp.float32).at[neg_inds.reshape(-1)].set(1.0)
    negmask = jnp.broadcast_to(negmask[None, None, :], (B, 1, N))

    kpt = jnp.concatenate([keypoints.transpose(0, 2, 1), negmask], axis=1)
    kpt = jnp.pad(kpt, ((0, 0), (0, 4), (0, npad - N)))

    clsf = class_ids.astype(jnp.float32)[..., None]
    bxt = jnp.concatenate([boxes[..., 0:3], clsf], axis=2)
    bxt = jnp.pad(bxt, ((0, 0), (0, _MPAD - nb), (0, 4)), constant_values=-1.0)

    clsrow = jnp.pad(class_ids.astype(jnp.float32)[:, None, :],
                     ((0, 0), (0, 7), (0, _MPAD - nb)), constant_values=-1.0)

    aux = jnp.zeros((8, 8), jnp.float32)
    aux = aux.at[0:3, 0:7].set(boxes.reshape(-1, 7)[0:_C])
    aux = aux.at[3:6, 0:3].set(anchor_sizes)
    aux = aux.at[6, 0:3].set(anchor_radii)

    clsf32, regf32 = pl.pallas_call(
        _body,
        grid=(B, npad // _T),
        in_specs=[
            pl.BlockSpec((8, 8), lambda b, n: (0, 0), memory_space=pltpu.SMEM),
            pl.BlockSpec((1, 8, _T), lambda b, n: (b, 0, n)),
            pl.BlockSpec((1, _MPAD, 8), lambda b, n: (b, 0, 0)),
            pl.BlockSpec((1, 8, _MPAD), lambda b, n: (b, 0, 0)),
        ],
        out_specs=[
            pl.BlockSpec((1, 8, _T), lambda b, n: (b, 0, n)),
            pl.BlockSpec((1, 32, _T), lambda b, n: (b, 0, n)),
        ],
        out_shape=[
            jax.ShapeDtypeStruct((B, 8, npad), jnp.float32),
            jax.ShapeDtypeStruct((B, 32, npad), jnp.float32),
        ],
    )(aux, kpt, bxt, clsrow)

    targets_cls = clsf32[:, :5, :N].transpose(0, 2, 1).astype(bool)
    targets_reg = regf32[:, :28, :N].transpose(0, 2, 1).reshape(B, N, 4, 7)
    return targets_cls, targets_reg
